# baseline (speedup 1.0000x reference)
import functools

import jax
import jax.numpy as jnp
from jax import lax
from jax.experimental import pallas as pl
from jax.experimental.pallas import tpu as pltpu
from jax.experimental.pallas import tpu_sc as plsc

B, N, M, C = 4096, 200, 50, 128
XROW = N * C
OROW = M * C
L = 16
NW = 32
BPW = B // NW


def _body(x_hbm, idx_hbm, out_hbm, xv, iv, ov):
    wid = lax.axis_index("s") * 2 + lax.axis_index("c")
    lane = lax.broadcasted_iota(jnp.int32, (L,), 0)

    def one_batch(k, carry):
        b = wid * BPW + k
        pltpu.sync_copy(x_hbm.at[b], xv)
        pltpu.sync_copy(idx_hbm.at[b], iv)

        def gather_step(j, c2):
            base = j * L
            idxv = iv[pl.ds(base, L)]
            col = lax.rem(base, C)
            gidx = idxv * C + col + lane
            ov[pl.ds(base, L)] = plsc.load_gather(xv, [gidx])
            return c2

        lax.fori_loop(0, OROW // L, gather_step, 0, unroll=8)
        pltpu.sync_copy(ov, out_hbm.at[b])
        return carry

    lax.fori_loop(0, BPW, one_batch, 0)


@jax.jit
def _gather_sc(x2, idx2):
    mesh = plsc.VectorSubcoreMesh(core_axis_name="c", subcore_axis_name="s")
    f = functools.partial(
        pl.kernel,
        out_type=jax.ShapeDtypeStruct((B, OROW), jnp.float32),
        mesh=mesh,
        scratch_types=[
            pltpu.VMEM((XROW,), jnp.float32),
            pltpu.VMEM((OROW,), jnp.int32),
            pltpu.VMEM((OROW,), jnp.float32),
        ],
        compiler_params=pltpu.CompilerParams(needs_layout_passes=False),
    )(_body)
    return f(x2, idx2)


def kernel(x, indices):
    x2 = x.reshape(B, XROW)
    idx2 = indices.astype(jnp.int32).reshape(B, OROW)
    out = _gather_sc(x2, idx2)
    return out.reshape(B, M, C)


# speedup vs baseline: 1.7707x; 1.7707x over previous
import functools

import jax
import jax.numpy as jnp
from jax import lax
from jax.experimental import pallas as pl
from jax.experimental.pallas import tpu as pltpu
from jax.experimental.pallas import tpu_sc as plsc

B, N, M, C = 4096, 200, 50, 128
XROW = N * C
OROW = M * C
L = 16
NW = 32
BPW = B // NW


def _body(x_hbm, idx_hbm, out_hbm,
          xv0, xv1, iv0, iv1, ov0, ov1,
          sx0, sx1, si0, si1, so0, so1):
    wid = lax.axis_index("s") * 2 + lax.axis_index("c")
    base_b = wid * BPW
    lane = lax.broadcasted_iota(jnp.int32, (L,), 0)
    xv, iv, ov = (xv0, xv1), (iv0, iv1), (ov0, ov1)
    sx, si, so = (sx0, sx1), (si0, si1), (so0, so1)

    for p in range(2):
        pltpu.async_copy(x_hbm.at[base_b + p], xv[p], sx[p])
        pltpu.async_copy(idx_hbm.at[base_b + p], iv[p], si[p])

    def gather_batch(src_x, src_i, dst_o):
        @plsc.parallel_loop(0, OROW // L, unroll=8)
        def _(j):
            bs = j * L
            idxv = src_i[pl.ds(bs, L)]
            col = lax.rem(bs, C)
            dst_o[pl.ds(bs, L)] = plsc.load_gather(
                src_x, [idxv * C + (col + lane)])

    def step(t, carry):
        for p in range(2):
            b = base_b + 2 * t + p
            pltpu.make_async_copy(x_hbm.at[b], xv[p], sx[p]).wait()
            pltpu.make_async_copy(idx_hbm.at[b], iv[p], si[p]).wait()

            @pl.when(t > 0)
            def _():
                pltpu.make_async_copy(ov[p], out_hbm.at[b - 2], so[p]).wait()

            gather_batch(xv[p], iv[p], ov[p])
            pltpu.async_copy(ov[p], out_hbm.at[b], so[p])

            @pl.when(t < BPW // 2 - 1)
            def _():
                pltpu.async_copy(x_hbm.at[b + 2], xv[p], sx[p])
                pltpu.async_copy(idx_hbm.at[b + 2], iv[p], si[p])
        return carry

    lax.fori_loop(0, BPW // 2, step, 0)
    for p in range(2):
        pltpu.make_async_copy(ov[p], out_hbm.at[base_b + BPW - 2 + p],
                              so[p]).wait()


@jax.jit
def _gather_sc(x2, idx2):
    mesh = plsc.VectorSubcoreMesh(core_axis_name="c", subcore_axis_name="s")
    f = functools.partial(
        pl.kernel,
        out_type=jax.ShapeDtypeStruct((B, OROW), jnp.float32),
        mesh=mesh,
        scratch_types=[
            pltpu.VMEM((XROW,), jnp.float32),
            pltpu.VMEM((XROW,), jnp.float32),
            pltpu.VMEM((OROW,), jnp.int32),
            pltpu.VMEM((OROW,), jnp.int32),
            pltpu.VMEM((OROW,), jnp.float32),
            pltpu.VMEM((OROW,), jnp.float32),
            pltpu.SemaphoreType.DMA,
            pltpu.SemaphoreType.DMA,
            pltpu.SemaphoreType.DMA,
            pltpu.SemaphoreType.DMA,
            pltpu.SemaphoreType.DMA,
            pltpu.SemaphoreType.DMA,
        ],
        compiler_params=pltpu.CompilerParams(needs_layout_passes=False),
    )(_body)
    return f(x2, idx2)


def kernel(x, indices):
    x2 = x.reshape(B, XROW)
    idx2 = indices.astype(jnp.int32).reshape(B, OROW)
    out = _gather_sc(x2, idx2)
    return out.reshape(B, M, C)


# speedup vs baseline: 3.9161x; 2.2116x over previous
import functools

import jax
import jax.numpy as jnp
from jax import lax
from jax.experimental import pallas as pl
from jax.experimental.pallas import tpu as pltpu
from jax.experimental.pallas import tpu_sc as plsc

B, N, M, C = 4096, 200, 50, 128
L = 16
NW = 32
BPW = B // NW
CHUNKS = (M * C) // L


def _body(x_hbm, idx_hbm, out_hbm,
          xv0, xv1, iv0, iv1, ov0, ov1,
          sx0, sx1, si0, si1, so0, so1):
    wid = lax.axis_index("s") * 2 + lax.axis_index("c")
    base_b = wid * BPW
    lane = lax.broadcasted_iota(jnp.int32, (L,), 0)
    xv, iv, ov = (xv0, xv1), (iv0, iv1), (ov0, ov1)
    sx, si, so = (sx0, sx1), (si0, si1), (so0, so1)

    for p in range(2):
        pltpu.async_copy(x_hbm.at[base_b + p], xv[p], sx[p])
        pltpu.async_copy(idx_hbm.at[base_b + p], iv[p], si[p])

    def gather_batch(src_x, src_i, dst_o):
        @plsc.parallel_loop(0, CHUNKS, unroll=8)
        def _(j):
            row = j // (C // L)
            cs = (j % (C // L)) * L
            idxv = src_i[row, pl.ds(cs, L)]
            dst_o[row, pl.ds(cs, L)] = plsc.load_gather(
                src_x, [idxv, cs + lane])

    def step(t, carry):
        for p in range(2):
            b = base_b + 2 * t + p
            pltpu.make_async_copy(x_hbm.at[b], xv[p], sx[p]).wait()
            pltpu.make_async_copy(idx_hbm.at[b], iv[p], si[p]).wait()

            @pl.when(t > 0)
            def _():
                pltpu.make_async_copy(ov[p], out_hbm.at[b - 2], so[p]).wait()

            gather_batch(xv[p], iv[p], ov[p])
            pltpu.async_copy(ov[p], out_hbm.at[b], so[p])

            @pl.when(t < BPW // 2 - 1)
            def _():
                pltpu.async_copy(x_hbm.at[b + 2], xv[p], sx[p])
                pltpu.async_copy(idx_hbm.at[b + 2], iv[p], si[p])
        return carry

    lax.fori_loop(0, BPW // 2, step, 0)
    for p in range(2):
        pltpu.make_async_copy(ov[p], out_hbm.at[base_b + BPW - 2 + p],
                              so[p]).wait()


@jax.jit
def _gather_sc(x, idx):
    mesh = plsc.VectorSubcoreMesh(core_axis_name="c", subcore_axis_name="s")
    f = functools.partial(
        pl.kernel,
        out_type=jax.ShapeDtypeStruct((B, M, C), jnp.float32),
        mesh=mesh,
        scratch_types=[
            pltpu.VMEM((N, C), jnp.float32),
            pltpu.VMEM((N, C), jnp.float32),
            pltpu.VMEM((M, C), jnp.int32),
            pltpu.VMEM((M, C), jnp.int32),
            pltpu.VMEM((M, C), jnp.float32),
            pltpu.VMEM((M, C), jnp.float32),
            pltpu.SemaphoreType.DMA,
            pltpu.SemaphoreType.DMA,
            pltpu.SemaphoreType.DMA,
            pltpu.SemaphoreType.DMA,
            pltpu.SemaphoreType.DMA,
            pltpu.SemaphoreType.DMA,
        ],
        compiler_params=pltpu.CompilerParams(needs_layout_passes=False),
    )(_body)
    return f(x, idx)


def kernel(x, indices):
    return _gather_sc(x, indices.astype(jnp.int32))
